# baseline (speedup 1.0000x reference)
import functools

import jax
import jax.numpy as jnp
from jax import lax
from jax.experimental import pallas as pl
from jax.experimental.pallas import tpu as pltpu
from jax.experimental.pallas import tpu_sc as plsc

BS = 16384
HALF = BS // 2
DIM = 128
NCLASS = 1000
CPAD = 1024
LANES = 16
NCORES = 2
NSUB = 16
NW = NCORES * NSUB
RPW = HALF // NW
CHUNK = 128
NCHUNKS = RPW // CHUNK
NVEC = DIM // LANES
CROWS = CPAD // NSUB
CLAST = NCLASS - (NSUB - 1) * CROWS

TCBLK = 512
NTCBLK = HALF // TCBLK


def _build_sc():
  mesh = plsc.VectorSubcoreMesh(core_axis_name="c", subcore_axis_name="s")

  @functools.partial(
      pl.kernel,
      mesh=mesh,
      out_type=jax.ShapeDtypeStruct((NW, LANES), jnp.float32),
      scratch_types=[
          pltpu.VMEM((NCHUNKS, CHUNK), jnp.int32),
          pltpu.VMEM((NCHUNKS, CHUNK, DIM), jnp.float32),
          pltpu.VMEM((CROWS, DIM), jnp.float32),
          pltpu.VMEM((CROWS, DIM), jnp.float32),
          pltpu.VMEM((LANES,), jnp.float32),
          pltpu.VMEM_SHARED((CPAD, DIM), jnp.float32),
          pltpu.SemaphoreType.DMA,
          pltpu.SemaphoreType.DMA,
          pltpu.SemaphoreType.DMA,
          pltpu.SemaphoreType.DMA,
          pltpu.SemaphoreType.DMA,
      ],
  )
  def k(feat_hbm, agents_hbm, labels_hbm, out_hbm,
        idx_v, feat_v, tbuf_v, abuf_v, res_v, table_sh,
        sem_f0, sem_f1, sem_l, sem_a, sem_s):
    sem_f = (sem_f0, sem_f1)
    cid = lax.axis_index("c")
    sid = lax.axis_index("s")
    wid = sid * NCORES + cid
    base = wid * RPW
    zero = jnp.zeros((LANES,), jnp.float32)

    lab_cp = pltpu.async_copy(labels_hbm.at[wid], idx_v, sem_l)
    feat_cps = [
        pltpu.async_copy(feat_hbm.at[pl.ds(base + j * CHUNK, CHUNK)],
                         feat_v.at[j], sem_f[j])
        for j in range(NCHUNKS)
    ]

    @pl.when(sid == NSUB - 1)
    def _():
      pltpu.async_copy(agents_hbm.at[pl.ds((NSUB - 1) * CROWS, CLAST)],
                       abuf_v.at[pl.ds(0, CLAST)], sem_a)

    @pl.when(sid != NSUB - 1)
    def _():
      pltpu.async_copy(agents_hbm.at[pl.ds(sid * CROWS, CROWS)],
                       abuf_v, sem_a)

    def zbody(r, _):
      for d in range(NVEC):
        tbuf_v[r, pl.ds(d * LANES, LANES)] = zero
      return 0
    lax.fori_loop(0, CROWS, zbody, 0)

    @pl.when(sid == NSUB - 1)
    def _():
      def ztail(r, _):
        for d in range(NVEC):
          abuf_v[r, pl.ds(d * LANES, LANES)] = zero
        return 0
      lax.fori_loop(CLAST, CROWS, ztail, 0)

    pltpu.sync_copy(tbuf_v, table_sh.at[pl.ds(sid * CROWS, CROWS)])
    lab_cp.wait()
    plsc.subcore_barrier()

    scat_cps = []
    for j in range(NCHUNKS):
      feat_cps[j].wait()
      scat_cps.append(
          pltpu.async_copy(feat_v.at[j], table_sh.at[idx_v.at[j]], sem_s,
                           add=True))
    for cp in scat_cps:
      cp.wait()
    plsc.subcore_barrier()

    pltpu.sync_copy(table_sh.at[pl.ds(sid * CROWS, CROWS)], tbuf_v)

    @pl.when(sid == NSUB - 1)
    def _():
      pltpu.make_async_copy(agents_hbm.at[pl.ds((NSUB - 1) * CROWS, CLAST)],
                            abuf_v.at[pl.ds(0, CLAST)], sem_a).wait()

    @pl.when(sid != NSUB - 1)
    def _():
      pltpu.make_async_copy(agents_hbm.at[pl.ds(sid * CROWS, CROWS)],
                            abuf_v, sem_a).wait()

    accs = tuple(jnp.zeros((LANES,), jnp.float32) for _ in range(NVEC))

    def body(r, acc):
      out = []
      for d in range(NVEC):
        t = tbuf_v[r, pl.ds(d * LANES, LANES)]
        a = abuf_v[r, pl.ds(d * LANES, LANES)]
        out.append(acc[d] + t * a)
      return tuple(out)

    accs = lax.fori_loop(0, CROWS, body, accs)

    total = accs[0]
    for d in range(1, NVEC):
      total = total + accs[d]
    res_v[...] = total
    pltpu.sync_copy(res_v, out_hbm.at[wid])

  return k


def _tc_body(feat_ref, lab_ref, agents_ref, out_ref):
  i = pl.program_id(0)
  lab = lab_ref[0, 0, :]
  cols = lax.broadcasted_iota(jnp.int32, (TCBLK, CPAD), 1)
  oh = (lab[:, None] == cols).astype(jnp.bfloat16)
  g = jnp.dot(oh, agents_ref[...],
              preferred_element_type=jnp.float32)
  prod = feat_ref[...] * g
  partial = jnp.sum(prod.reshape(TCBLK // 8, 8, DIM), axis=0)

  @pl.when(i == 0)
  def _():
    out_ref[...] = partial

  @pl.when(i != 0)
  def _():
    out_ref[...] += partial


_sc_kernel = _build_sc()

_tc_kernel = pl.pallas_call(
    _tc_body,
    grid=(NTCBLK,),
    in_specs=[
        pl.BlockSpec((TCBLK, DIM), lambda i: (i + NTCBLK, 0)),
        pl.BlockSpec((1, 1, TCBLK), lambda i: (i + NTCBLK, 0, 0)),
        pl.BlockSpec((CPAD, DIM), lambda i: (0, 0)),
    ],
    out_specs=pl.BlockSpec((8, DIM), lambda i: (0, 0)),
    out_shape=jax.ShapeDtypeStruct((8, DIM), jnp.float32),
)


@jax.jit
def kernel(features, agents, labels):
  labels_i32 = labels.astype(jnp.int32)
  labels_sc = labels_i32[:HALF].reshape(NW, NCHUNKS, CHUNK)
  labels_tc = labels_i32.reshape(2 * NTCBLK, 1, TCBLK)
  agents_bf = jnp.concatenate(
      [agents, jnp.zeros((CPAD - NCLASS, DIM), agents.dtype)],
      axis=0).astype(jnp.bfloat16)
  sc_partials = _sc_kernel(features, agents, labels_sc)
  tc_partial = _tc_kernel(features, labels_tc, agents_bf)
  total = sc_partials.sum() + tc_partial.sum()
  return 1.0 - total / BS


# speedup vs baseline: 1.2138x; 1.2138x over previous
import functools

import jax
import jax.numpy as jnp
from jax import lax
from jax.experimental import pallas as pl
from jax.experimental.pallas import tpu as pltpu
from jax.experimental.pallas import tpu_sc as plsc

BS = 16384
DIM = 128
NCLASS = 1000
CPAD = 1024
LANES = 16
NCORES = 2
NSUB = 16
NW = NCORES * NSUB
RPW = BS // NW
CHUNK = 128
NCHUNKS = RPW // CHUNK
NVEC = DIM // LANES
CROWS = CPAD // NSUB
CLAST = NCLASS - (NSUB - 1) * CROWS


def _build():
  mesh = plsc.VectorSubcoreMesh(core_axis_name="c", subcore_axis_name="s")

  @functools.partial(
      pl.kernel,
      mesh=mesh,
      out_type=jax.ShapeDtypeStruct((NW, LANES), jnp.float32),
      scratch_types=[
          pltpu.VMEM((NCHUNKS, CHUNK), jnp.int32),
          pltpu.VMEM((NCHUNKS, CHUNK, DIM), jnp.float32),
          pltpu.VMEM((CROWS, DIM), jnp.float32),
          pltpu.VMEM((CROWS, DIM), jnp.float32),
          pltpu.VMEM((CROWS, DIM), jnp.float32),
          pltpu.VMEM((LANES,), jnp.float32),
          pltpu.VMEM_SHARED((2, CPAD, DIM), jnp.float32),
          pltpu.SemaphoreType.DMA,
          pltpu.SemaphoreType.DMA,
          pltpu.SemaphoreType.DMA,
          pltpu.SemaphoreType.DMA,
          pltpu.SemaphoreType.DMA,
          pltpu.SemaphoreType.DMA,
          pltpu.SemaphoreType.DMA,
      ],
  )
  def k(feat_hbm, agents_hbm, labels_hbm, out_hbm,
        idx_v, feat_v, tbuf_v, abuf_v, tbuf2_v, res_v, table_sh,
        sem_f0, sem_f1, sem_f2, sem_f3, sem_l, sem_a, sem_s):
    sem_f = (sem_f0, sem_f1, sem_f2, sem_f3)
    cid = lax.axis_index("c")
    sid = lax.axis_index("s")
    wid = sid * NCORES + cid
    base = wid * RPW
    zero = jnp.zeros((LANES,), jnp.float32)

    lab_cp = pltpu.async_copy(labels_hbm.at[wid], idx_v, sem_l)
    feat_cps = [
        pltpu.async_copy(feat_hbm.at[pl.ds(base + j * CHUNK, CHUNK)],
                         feat_v.at[j], sem_f[j])
        for j in range(NCHUNKS)
    ]

    @pl.when(sid == NSUB - 1)
    def _():
      pltpu.async_copy(agents_hbm.at[pl.ds((NSUB - 1) * CROWS, CLAST)],
                       abuf_v.at[pl.ds(0, CLAST)], sem_a)

    @pl.when(sid != NSUB - 1)
    def _():
      pltpu.async_copy(agents_hbm.at[pl.ds(sid * CROWS, CROWS)],
                       abuf_v, sem_a)

    def zbody(r, _):
      for d in range(NVEC):
        tbuf_v[r, pl.ds(d * LANES, LANES)] = zero
      return 0
    lax.fori_loop(0, CROWS, zbody, 0)

    @pl.when(sid == NSUB - 1)
    def _():
      def ztail(r, _):
        for d in range(NVEC):
          abuf_v[r, pl.ds(d * LANES, LANES)] = zero
        return 0
      lax.fori_loop(CLAST, CROWS, ztail, 0)

    pltpu.sync_copy(tbuf_v, table_sh.at[0].at[pl.ds(sid * CROWS, CROWS)])
    pltpu.sync_copy(tbuf_v, table_sh.at[1].at[pl.ds(sid * CROWS, CROWS)])
    lab_cp.wait()
    plsc.subcore_barrier()

    def scatter_phase(tref):
      cps = []
      for j in range(NCHUNKS):
        feat_cps[j].wait()
        cps.append(
            pltpu.async_copy(feat_v.at[j], tref.at[idx_v.at[j]], sem_s,
                             add=True))
      for cp in cps:
        cp.wait()

    @pl.when(sid % 2 == 0)
    def _():
      scatter_phase(table_sh.at[0])

    @pl.when(sid % 2 == 1)
    def _():
      scatter_phase(table_sh.at[1])

    plsc.subcore_barrier()

    pltpu.sync_copy(table_sh.at[0].at[pl.ds(sid * CROWS, CROWS)], tbuf_v)
    pltpu.sync_copy(table_sh.at[1].at[pl.ds(sid * CROWS, CROWS)], tbuf2_v)

    @pl.when(sid == NSUB - 1)
    def _():
      pltpu.make_async_copy(agents_hbm.at[pl.ds((NSUB - 1) * CROWS, CLAST)],
                            abuf_v.at[pl.ds(0, CLAST)], sem_a).wait()

    @pl.when(sid != NSUB - 1)
    def _():
      pltpu.make_async_copy(agents_hbm.at[pl.ds(sid * CROWS, CROWS)],
                            abuf_v, sem_a).wait()

    accs = tuple(jnp.zeros((LANES,), jnp.float32) for _ in range(NVEC))

    def body(r, acc):
      out = []
      for d in range(NVEC):
        t = tbuf_v[r, pl.ds(d * LANES, LANES)] + tbuf2_v[r, pl.ds(d * LANES, LANES)]
        a = abuf_v[r, pl.ds(d * LANES, LANES)]
        out.append(acc[d] + t * a)
      return tuple(out)

    accs = lax.fori_loop(0, CROWS, body, accs)

    total = accs[0]
    for d in range(1, NVEC):
      total = total + accs[d]
    res_v[...] = total
    pltpu.sync_copy(res_v, out_hbm.at[wid])

  return k


_partials_kernel = _build()


@jax.jit
def kernel(features, agents, labels):
  labels_i32 = labels.astype(jnp.int32).reshape(NW, NCHUNKS, CHUNK)
  partials = _partials_kernel(features, agents, labels_i32)
  return 1.0 - partials.sum() / BS


# speedup vs baseline: 1.2334x; 1.0162x over previous
import functools

import jax
import jax.numpy as jnp
from jax import lax
from jax.experimental import pallas as pl
from jax.experimental.pallas import tpu as pltpu
from jax.experimental.pallas import tpu_sc as plsc

BS = 16384
DIM = 128
NCLASS = 1000
CPAD = 1024
LANES = 16
NCORES = 2
NSUB = 16
NW = NCORES * NSUB
RPW = BS // NW
CHUNK = 128
NCHUNKS = RPW // CHUNK
NVEC = DIM // LANES
CROWS = CPAD // NSUB
CLAST = NCLASS - (NSUB - 1) * CROWS


def _build():
  mesh = plsc.VectorSubcoreMesh(core_axis_name="c", subcore_axis_name="s")

  @functools.partial(
      pl.kernel,
      mesh=mesh,
      out_type=jax.ShapeDtypeStruct((NW, LANES), jnp.float32),
      scratch_types=[
          pltpu.VMEM((NCHUNKS, CHUNK), jnp.int32),
          pltpu.VMEM((NCHUNKS, CHUNK, DIM), jnp.float32),
          pltpu.VMEM((CROWS, DIM), jnp.float32),
          pltpu.VMEM((CROWS, DIM), jnp.float32),
          pltpu.VMEM((LANES,), jnp.float32),
          pltpu.VMEM_SHARED((CPAD, DIM), jnp.float32),
          pltpu.SemaphoreType.DMA,
          pltpu.SemaphoreType.DMA,
          pltpu.SemaphoreType.DMA,
          pltpu.SemaphoreType.DMA,
          pltpu.SemaphoreType.DMA,
          pltpu.SemaphoreType.DMA,
          pltpu.SemaphoreType.DMA,
          pltpu.SemaphoreType.DMA,
      ],
  )
  def k(feat_hbm, agents_hbm, labels_hbm, zeros_hbm, out_hbm,
        idx_v, feat_v, tbuf_v, abuf_v, res_v, table_sh,
        sem_f0, sem_f1, sem_f2, sem_f3, sem_l, sem_a, sem_s, sem_z):
    sem_f = (sem_f0, sem_f1, sem_f2, sem_f3)
    cid = lax.axis_index("c")
    sid = lax.axis_index("s")
    wid = sid * NCORES + cid
    base = wid * RPW
    zero = jnp.zeros((LANES,), jnp.float32)

    lab_cp = pltpu.async_copy(labels_hbm.at[wid], idx_v, sem_l)
    feat_cps = [
        pltpu.async_copy(feat_hbm.at[pl.ds(base + j * CHUNK, CHUNK)],
                         feat_v.at[j], sem_f[j])
        for j in range(NCHUNKS)
    ]

    @pl.when(sid == NSUB - 1)
    def _():
      pltpu.async_copy(agents_hbm.at[pl.ds((NSUB - 1) * CROWS, CLAST)],
                       abuf_v.at[pl.ds(0, CLAST)], sem_a)

    @pl.when(sid != NSUB - 1)
    def _():
      pltpu.async_copy(agents_hbm.at[pl.ds(sid * CROWS, CROWS)],
                       abuf_v, sem_a)

    zcp = pltpu.async_copy(zeros_hbm.at[pl.ds(sid * CROWS, CROWS)],
                           table_sh.at[pl.ds(sid * CROWS, CROWS)], sem_z)

    @pl.when(sid == NSUB - 1)
    def _():
      def ztail(r, _):
        for d in range(NVEC):
          abuf_v[r, pl.ds(d * LANES, LANES)] = zero
        return 0
      lax.fori_loop(CLAST, CROWS, ztail, 0)

    zcp.wait()
    lab_cp.wait()
    plsc.subcore_barrier()

    scat_cps = []
    for j in range(NCHUNKS):
      feat_cps[j].wait()
      scat_cps.append(
          pltpu.async_copy(feat_v.at[j], table_sh.at[idx_v.at[j]], sem_s,
                           add=True))
    for cp in scat_cps:
      cp.wait()
    plsc.subcore_barrier()

    pltpu.sync_copy(table_sh.at[pl.ds(sid * CROWS, CROWS)], tbuf_v)

    @pl.when(sid == NSUB - 1)
    def _():
      pltpu.make_async_copy(agents_hbm.at[pl.ds((NSUB - 1) * CROWS, CLAST)],
                            abuf_v.at[pl.ds(0, CLAST)], sem_a).wait()

    @pl.when(sid != NSUB - 1)
    def _():
      pltpu.make_async_copy(agents_hbm.at[pl.ds(sid * CROWS, CROWS)],
                            abuf_v, sem_a).wait()

    accs = tuple(jnp.zeros((LANES,), jnp.float32) for _ in range(NVEC))

    def body(r, acc):
      out = []
      for d in range(NVEC):
        t = tbuf_v[r, pl.ds(d * LANES, LANES)]
        a = abuf_v[r, pl.ds(d * LANES, LANES)]
        out.append(acc[d] + t * a)
      return tuple(out)

    accs = lax.fori_loop(0, CROWS, body, accs)

    total = accs[0]
    for d in range(1, NVEC):
      total = total + accs[d]
    res_v[...] = total
    pltpu.sync_copy(res_v, out_hbm.at[wid])

  return k


_partials_kernel = _build()


@jax.jit
def kernel(features, agents, labels):
  labels_i32 = labels.astype(jnp.int32).reshape(NW, NCHUNKS, CHUNK)
  zeros = jnp.zeros((CPAD, DIM), jnp.float32)
  partials = _partials_kernel(features, agents, labels_i32, zeros)
  return 1.0 - partials.sum() / BS


# speedup vs baseline: 1.2391x; 1.0046x over previous
import functools

import jax
import jax.numpy as jnp
from jax import lax
from jax.experimental import pallas as pl
from jax.experimental.pallas import tpu as pltpu
from jax.experimental.pallas import tpu_sc as plsc

BS = 16384
DIM = 128
NCLASS = 1000
CPAD = 1024
LANES = 16
NCORES = 2
NSUB = 16
NW = NCORES * NSUB
RPW = BS // NW
CHUNK = 128
NCHUNKS = RPW // CHUNK
NVEC = DIM // LANES
CROWS = CPAD // NSUB
CLAST = NCLASS - (NSUB - 1) * CROWS


def _build():
  mesh = plsc.VectorSubcoreMesh(core_axis_name="c", subcore_axis_name="s")

  @functools.partial(
      pl.kernel,
      mesh=mesh,
      out_type=jax.ShapeDtypeStruct((NW, LANES), jnp.float32),
      scratch_types=[
          pltpu.VMEM((NCHUNKS, CHUNK), jnp.int32),
          pltpu.VMEM((NCHUNKS, CHUNK, DIM), jnp.float32),
          pltpu.VMEM((CROWS, DIM), jnp.float32),
          pltpu.VMEM((CROWS, DIM), jnp.float32),
          pltpu.VMEM((LANES,), jnp.float32),
          pltpu.VMEM_SHARED((CPAD, DIM), jnp.float32),
          pltpu.SemaphoreType.DMA,
          pltpu.SemaphoreType.DMA,
          pltpu.SemaphoreType.DMA,
          pltpu.SemaphoreType.DMA,
          pltpu.SemaphoreType.DMA,
          pltpu.SemaphoreType.DMA,
          pltpu.SemaphoreType.DMA,
      ],
  )
  def k(feat_hbm, agents_hbm, labels_hbm, out_hbm,
        idx_v, feat_v, tbuf_v, abuf_v, res_v, table_sh,
        sem_f0, sem_f1, sem_f2, sem_f3, sem_l, sem_a, sem_s):
    sem_f = (sem_f0, sem_f1, sem_f2, sem_f3)
    cid = lax.axis_index("c")
    sid = lax.axis_index("s")
    wid = sid * NCORES + cid
    base = wid * RPW
    zero = jnp.zeros((LANES,), jnp.float32)

    lab_cp = pltpu.async_copy(labels_hbm.at[wid], idx_v, sem_l)
    feat_cps = [
        pltpu.async_copy(feat_hbm.at[pl.ds(base + j * CHUNK, CHUNK)],
                         feat_v.at[j], sem_f[j])
        for j in range(NCHUNKS)
    ]

    @pl.when(sid == NSUB - 1)
    def _():
      pltpu.async_copy(agents_hbm.at[pl.ds((NSUB - 1) * CROWS, CLAST)],
                       abuf_v.at[pl.ds(0, CLAST)], sem_a)

    @pl.when(sid != NSUB - 1)
    def _():
      pltpu.async_copy(agents_hbm.at[pl.ds(sid * CROWS, CROWS)],
                       abuf_v, sem_a)

    def zbody(r, _):
      for d in range(NVEC):
        tbuf_v[r, pl.ds(d * LANES, LANES)] = zero
      return 0
    lax.fori_loop(0, CROWS, zbody, 0)

    @pl.when(sid == NSUB - 1)
    def _():
      def ztail(r, _):
        for d in range(NVEC):
          abuf_v[r, pl.ds(d * LANES, LANES)] = zero
        return 0
      lax.fori_loop(CLAST, CROWS, ztail, 0)

    pltpu.sync_copy(tbuf_v, table_sh.at[pl.ds(sid * CROWS, CROWS)])
    lab_cp.wait()
    plsc.subcore_barrier()

    scat_cps = []
    for j in range(NCHUNKS):
      feat_cps[j].wait()
      scat_cps.append(
          pltpu.async_copy(feat_v.at[j], table_sh.at[idx_v.at[j]], sem_s,
                           add=True))
    for cp in scat_cps:
      cp.wait()
    plsc.subcore_barrier()

    pltpu.sync_copy(table_sh.at[pl.ds(sid * CROWS, CROWS)], tbuf_v)

    @pl.when(sid == NSUB - 1)
    def _():
      pltpu.make_async_copy(agents_hbm.at[pl.ds((NSUB - 1) * CROWS, CLAST)],
                            abuf_v.at[pl.ds(0, CLAST)], sem_a).wait()

    @pl.when(sid != NSUB - 1)
    def _():
      pltpu.make_async_copy(agents_hbm.at[pl.ds(sid * CROWS, CROWS)],
                            abuf_v, sem_a).wait()

    accs = tuple(jnp.zeros((LANES,), jnp.float32) for _ in range(NVEC))

    def body(r, acc):
      out = []
      for d in range(NVEC):
        t = tbuf_v[r, pl.ds(d * LANES, LANES)]
        a = abuf_v[r, pl.ds(d * LANES, LANES)]
        out.append(acc[d] + t * a)
      return tuple(out)

    accs = lax.fori_loop(0, CROWS, body, accs)

    total = accs[0]
    for d in range(1, NVEC):
      total = total + accs[d]
    res_v[...] = total
    pltpu.sync_copy(res_v, out_hbm.at[wid])

  return k


_partials_kernel = _build()


@jax.jit
def kernel(features, agents, labels):
  labels_i32 = labels.astype(jnp.int32).reshape(NW, NCHUNKS, CHUNK)
  partials = _partials_kernel(features, agents, labels_i32)
  return 1.0 - partials.sum() / BS
